# baseline (speedup 1.0000x reference)
import functools

import jax
import jax.numpy as jnp
from jax import lax
from jax.experimental import pallas as pl
from jax.experimental.pallas import tpu as pltpu
from jax.experimental.pallas import tpu_sc as plsc

VOCAB = 1000000
D = 64
B = 4096 * 200
NC, NS = 2, 16
NW = NC * NS
B_PER_W = B // NW
CH = 128
NBUF = 4
N_CH = B_PER_W // CH
NG = N_CH // NBUF

_mesh = plsc.VectorSubcoreMesh(
    core_axis_name="c", subcore_axis_name="s", num_cores=NC, num_subcores=NS
)


@functools.partial(
    pl.kernel,
    out_type=jax.ShapeDtypeStruct((B, D), jnp.float32),
    mesh=_mesh,
    compiler_params=pltpu.CompilerParams(use_tc_tiling_on_sc=False),
    scratch_types=[
        pltpu.VMEM((B_PER_W,), jnp.int32),
        pltpu.VMEM((NBUF, CH, D), jnp.float32),
    ]
    + [pltpu.SemaphoreType.DMA] * NBUF
    + [pltpu.SemaphoreType.DMA] * NBUF,
)
def _embed_sc(x_hbm, table_hbm, out_hbm, idx_v, rows_v, *sems):
    gsems = sems[:NBUF]
    osems = sems[NBUF:]
    wid = lax.axis_index("s") * NC + lax.axis_index("c")
    base = pl.multiple_of(wid * B_PER_W, B_PER_W)

    pltpu.sync_copy(x_hbm.at[pl.ds(base, B_PER_W)], idx_v)

    def group(g, _):
        for b in range(NBUF):
            c = g * NBUF + b
            off = pl.multiple_of(base + c * CH, CH)

            @pl.when(g > 0)
            def _drain():
                pltpu.make_async_copy(
                    rows_v.at[b], out_hbm.at[pl.ds(off, CH)], osems[b]
                ).wait()

            pltpu.make_async_copy(
                table_hbm.at[idx_v.at[pl.ds(c * CH, CH)]],
                rows_v.at[b],
                gsems[b],
            ).start()

        for b in range(NBUF):
            c = g * NBUF + b
            off = pl.multiple_of(base + c * CH, CH)
            pltpu.make_async_copy(
                table_hbm.at[idx_v.at[pl.ds(c * CH, CH)]],
                rows_v.at[b],
                gsems[b],
            ).wait()
            pltpu.make_async_copy(
                rows_v.at[b], out_hbm.at[pl.ds(off, CH)], osems[b]
            ).start()
        return _

    lax.fori_loop(0, NG, group, 0)

    for b in range(NBUF):
        c = (NG - 1) * NBUF + b
        off = pl.multiple_of(base + c * CH, CH)
        pltpu.make_async_copy(
            rows_v.at[b], out_hbm.at[pl.ds(off, CH)], osems[b]
        ).wait()


@jax.jit
def kernel(x, table):
    out = _embed_sc(x.reshape(-1), table)
    return out.reshape(x.shape[0], x.shape[1], D)


# speedup vs baseline: 1.0285x; 1.0285x over previous
import functools

import jax
import jax.numpy as jnp
from jax import lax
from jax.experimental import pallas as pl
from jax.experimental.pallas import tpu as pltpu
from jax.experimental.pallas import tpu_sc as plsc

VOCAB = 1000000
D = 64
B = 4096 * 200
NC, NS = 2, 16
NW = NC * NS
B_PER_W = B // NW
CH = 128
NBUF = 4
N_CH = B_PER_W // CH
NG = N_CH // NBUF

_mesh = plsc.VectorSubcoreMesh(
    core_axis_name="c", subcore_axis_name="s", num_cores=NC, num_subcores=NS
)


@functools.partial(
    pl.kernel,
    out_type=jax.ShapeDtypeStruct((B, D), jnp.float32),
    mesh=_mesh,
    compiler_params=pltpu.CompilerParams(use_tc_tiling_on_sc=False),
    scratch_types=[
        pltpu.VMEM((B_PER_W,), jnp.int32),
        pltpu.VMEM((NBUF, CH, D), jnp.float32),
    ]
    + [pltpu.SemaphoreType.DMA] * NBUF
    + [pltpu.SemaphoreType.DMA] * NBUF,
)
def _embed_sc(x_hbm, table_hbm, out_hbm, idx_v, rows_v, *sems):
    gsems = sems[:NBUF]
    osems = sems[NBUF:]
    wid = lax.axis_index("s") * NC + lax.axis_index("c")
    base = pl.multiple_of(wid * B_PER_W, B_PER_W)

    pltpu.sync_copy(x_hbm.at[pl.ds(base, B_PER_W)], idx_v)

    def group(g, _):
        for b in range(NBUF):
            c = g * NBUF + b
            off = pl.multiple_of(base + c * CH, CH)

            @pl.when(g > 0)
            def _drain():
                pltpu.make_async_copy(
                    rows_v.at[b], out_hbm.at[pl.ds(off, CH)], osems[b]
                ).wait()

            pltpu.make_async_copy(
                table_hbm.at[idx_v.at[pl.ds(c * CH, CH)]],
                rows_v.at[b],
                gsems[b],
            ).start()

        for b in range(NBUF):
            c = g * NBUF + b
            off = pl.multiple_of(base + c * CH, CH)
            pltpu.make_async_copy(
                table_hbm.at[idx_v.at[pl.ds(c * CH, CH)]],
                rows_v.at[b],
                gsems[b],
            ).wait()
            pltpu.make_async_copy(
                rows_v.at[b], out_hbm.at[pl.ds(off, CH)], osems[b]
            ).start()
        return _

    lax.fori_loop(0, NG, group, 0)

    for b in range(NBUF):
        c = (NG - 1) * NBUF + b
        off = pl.multiple_of(base + c * CH, CH)
        pltpu.make_async_copy(
            rows_v.at[b], out_hbm.at[pl.ds(off, CH)], osems[b]
        ).wait()


@jax.jit
def kernel(x, table):
    xt = jnp.swapaxes(x, 0, 1).reshape(-1)
    out = _embed_sc(xt, table)
    return jnp.swapaxes(out.reshape(x.shape[1], x.shape[0], D), 0, 1)
